# baseline (speedup 1.0000x reference)
import jax
import jax.numpy as jnp
from jax.experimental import pallas as pl
from jax.experimental.pallas import tpu as pltpu


_BLOCK_N = 4096


def _scoring_kernel(x_ref, wa_ref, wb_ref, b_ref, o_ref):
    x = x_ref[...]
    dn = (((1,), (1,)), ((), ()))
    o_ref[:, :_BLOCK_N] = (
        jax.lax.dot_general(x, wa_ref[...], dn,
                            preferred_element_type=jnp.float32)
        + b_ref[:, :_BLOCK_N]
    )
    o_ref[:, _BLOCK_N:] = (
        jax.lax.dot_general(x, wb_ref[...], dn,
                            preferred_element_type=jnp.float32)
        + b_ref[:, _BLOCK_N:]
    )


def kernel(x, y, freeze_flag, W, b):
    del y, freeze_flag
    B, D = x.shape
    N = W.shape[0]
    b2 = b.reshape(1, N)
    step = 2 * _BLOCK_N
    last = (N - 1) // _BLOCK_N
    out = pl.pallas_call(
        _scoring_kernel,
        grid=(pl.cdiv(N, step),),
        in_specs=[
            pl.BlockSpec((B, D), lambda i: (0, 0)),
            pl.BlockSpec((_BLOCK_N, D), lambda i: (2 * i, 0)),
            pl.BlockSpec((_BLOCK_N, D),
                         lambda i: (jnp.minimum(2 * i + 1, last), 0)),
            pl.BlockSpec((1, step), lambda i: (0, i)),
        ],
        out_specs=pl.BlockSpec((B, step), lambda i: (0, i)),
        out_shape=jax.ShapeDtypeStruct((B, N), jnp.float32),
        compiler_params=pltpu.CompilerParams(
            dimension_semantics=("arbitrary",),
        ),
    )(x, W, W, b2)
    return out
